# baseline (speedup 1.0000x reference)
import functools

import jax
import jax.numpy as jnp
from jax import lax
from jax.experimental import pallas as pl
from jax.experimental.pallas import tpu as pltpu
from jax.experimental.pallas import tpu_sc as plsc

B, H, W, C = 4, 384, 384, 96
HW = H * W
P = B * HW
NC, NS, L = 2, 16, 16
NW = NC * NS
PIX_PER_W = P // NW
K = 128
NCHUNK = PIX_PER_W // K
SG = 8
NSUPER = NCHUNK // SG
WPI = HW // PIX_PER_W
L2 = 2 * L


def _body(im_hbm, grid_hbm, out_hbm,
          gx_v, gy_v, wx_v, wy_v, idx_v,
          q_v, out_v, sem_g, sem_out, sem_grid):
    wid = lax.axis_index("s") * NC + lax.axis_index("c")
    b = wid // WPI
    pix0 = (wid % WPI) * PIX_PER_W
    row_base = b * HW

    def grid_fetch(s, buf):
        q0 = pix0 + s * (SG * K)
        pltpu.async_copy(grid_hbm.at[b, 0, pl.ds(q0, SG * K)],
                         gx_v.at[buf], sem_grid)
        pltpu.async_copy(grid_hbm.at[b, 1, pl.ds(q0, SG * K)],
                         gy_v.at[buf], sem_grid)

    def grid_wait():
        pltpu.make_async_copy(grid_hbm.at[b, 0, pl.ds(0, SG * K)],
                              gx_v.at[0], sem_grid).wait()
        pltpu.make_async_copy(grid_hbm.at[b, 1, pl.ds(0, SG * K)],
                              gy_v.at[0], sem_grid).wait()

    def prep_and_fire(g, gbuf, qbuf):
        off = (g % SG) * K
        for i in range(K // L):
            sl = pl.ds(off + i * L, L)
            so = pl.ds(i * L, L)
            gx = (gx_v[gbuf, sl] + 1.0) * (W * 0.5)
            gy = (gy_v[gbuf, sl] + 1.0) * (H * 0.5)
            x0 = jnp.minimum(gx.astype(jnp.int32), W - 2)
            y0 = jnp.minimum(gy.astype(jnp.int32), H - 2)
            wx_v[qbuf, so] = gx - x0.astype(jnp.float32)
            wy_v[qbuf, so] = gy - y0.astype(jnp.float32)
            i00 = row_base + y0 * W + x0
            idx_v[qbuf, 0, so] = i00
            idx_v[qbuf, 1, so] = i00 + W
            idx_v[qbuf, 2, so] = i00 + 1
            idx_v[qbuf, 3, so] = i00 + (W + 1)
        for r in range(4):
            pltpu.async_copy(im_hbm.at[idx_v.at[qbuf, r]],
                             q_v.at[qbuf, r], sem_g)

    def gather_wait(qbuf):
        for r in range(4):
            pltpu.make_async_copy(im_hbm.at[idx_v.at[qbuf, r]],
                                  q_v.at[qbuf, r], sem_g).wait()

    def combine(g, qbuf):
        def pix_group(i, carry):
            base = i * L
            wxg = wx_v[qbuf, pl.ds(base, L)]
            wyg = wy_v[qbuf, pl.ds(base, L)]
            for j in range(L):
                p = base + j
                wxf = jnp.broadcast_to(wxg[j], (L,))
                wyf = jnp.broadcast_to(wyg[j], (L,))
                wx = plsc.pack(wxf, wxf, format=plsc.PackFormat.INTERLEAVED)
                wy = plsc.pack(wyf, wyf, format=plsc.PackFormat.INTERLEAVED)
                for cg in range(C // L2):
                    cs = pl.ds(cg * L2, L2)
                    q00 = q_v[qbuf, 0, p, cs]
                    q01 = q_v[qbuf, 1, p, cs]
                    q10 = q_v[qbuf, 2, p, cs]
                    q11 = q_v[qbuf, 3, p, cs]
                    top = q00 + wx * (q10 - q00)
                    bot = q01 + wx * (q11 - q01)
                    out_v[qbuf, p, cs] = top + wy * (bot - top)
            return carry

        lax.fori_loop(0, K // L, pix_group, 0, unroll=False)
        pltpu.async_copy(out_v.at[qbuf],
                         out_hbm.at[pl.ds(row_base + pix0 + g * K, K)],
                         sem_out)

    def out_wait(g, qbuf):
        pltpu.make_async_copy(
            out_v.at[qbuf],
            out_hbm.at[pl.ds(row_base + pix0 + g * K, K)],
            sem_out).wait()

    grid_fetch(0, 0)
    grid_wait()
    prep_and_fire(0, 0, 0)

    def super_step(s, carry):
        @pl.when(s + 1 < NSUPER)
        def _():
            grid_fetch(s + 1, (s + 1) % 2)

        def pair(h, carry2):
            g0 = s * SG + 2 * h
            prep_and_fire(g0 + 1, s % 2, 1)

            @pl.when(g0 >= 2)
            def _():
                out_wait(g0 - 2, 0)
            gather_wait(0)
            combine(g0, 0)

            @pl.when(g0 + 2 < NCHUNK)
            def _():
                last_pair = h == SG // 2 - 1

                @pl.when(last_pair)
                def _():
                    grid_wait()
                    prep_and_fire(g0 + 2, (s + 1) % 2, 0)

                @pl.when(jnp.logical_not(last_pair))
                def _():
                    prep_and_fire(g0 + 2, s % 2, 0)

            @pl.when(g0 >= 2)
            def _():
                out_wait(g0 - 1, 1)
            gather_wait(1)
            combine(g0 + 1, 1)
            return carry2

        lax.fori_loop(0, SG // 2, pair, 0, unroll=False)
        return carry

    lax.fori_loop(0, NSUPER, super_step, 0, unroll=False)
    out_wait(NCHUNK - 2, 0)
    out_wait(NCHUNK - 1, 1)


@jax.jit
def kernel(im, grid):
    im_flat = im.reshape(P, C).astype(jnp.bfloat16)
    grid_flat = grid.reshape(B, 2, HW)
    run = pl.kernel(
        _body,
        out_type=jax.ShapeDtypeStruct((P, C), jnp.bfloat16),
        mesh=plsc.VectorSubcoreMesh(core_axis_name="c", subcore_axis_name="s"),
        scratch_types=[
            pltpu.VMEM((2, SG * K), jnp.float32),
            pltpu.VMEM((2, SG * K), jnp.float32),
            pltpu.VMEM((2, K), jnp.float32),
            pltpu.VMEM((2, K), jnp.float32),
            pltpu.VMEM((2, 4, K), jnp.int32),
            pltpu.VMEM((2, 4, K, C), jnp.bfloat16),
            pltpu.VMEM((2, K, C), jnp.bfloat16),
            pltpu.SemaphoreType.DMA,
            pltpu.SemaphoreType.DMA,
            pltpu.SemaphoreType.DMA,
        ],
        compiler_params=pltpu.CompilerParams(use_tc_tiling_on_sc=False,
                                             needs_layout_passes=False),
    )
    out = run(im_flat, grid_flat)
    return out.astype(jnp.float32).reshape(B, H, W, C)
